# baseline (speedup 1.0000x reference)
import functools

import jax
import jax.numpy as jnp
from jax import lax
from jax.experimental import pallas as pl
from jax.experimental.pallas import tpu as pltpu
from jax.experimental.pallas import tpu_sc as plsc

N = 10000
E = 320000
DN = 128
DE = 16
DG = 128
H1 = 256
H2 = 128


_BE = 2000
_EGRID = E // _BE


def _edge_mlp_a_body(edges_ref, we1a_ref, be1a_ref, e1a_ref):
    x = edges_ref[...]
    e1a_ref[...] = jnp.maximum(
        jnp.dot(x, we1a_ref[...], preferred_element_type=jnp.float32)
        + be1a_ref[...], 0.0)


def _edge_mlp_a(edges, W_e1a, b_e1a):
    full = lambda r, c: pl.BlockSpec((r, c), lambda i: (0, 0))
    return pl.pallas_call(
        _edge_mlp_a_body,
        grid=(_EGRID,),
        in_specs=[
            pl.BlockSpec((_BE, DE), lambda i: (i, 0)),
            full(DE, H2), full(1, H2),
        ],
        out_specs=[pl.BlockSpec((_BE, H2), lambda i: (i, 0))],
        out_shape=[jax.ShapeDtypeStruct((E, H2), jnp.float32)],
    )(edges, W_e1a, b_e1a.reshape(1, H2))[0]


def _edge_mlp_c_body(edges_ref, we1_ref, be1_ref, we2_ref, be2_ref,
                     e2_ref, cs1_ref, cs2_ref):
    i = pl.program_id(0)
    x = edges_ref[...]
    e1 = jnp.maximum(
        jnp.dot(x, we1_ref[...], preferred_element_type=jnp.float32)
        + be1_ref[...], 0.0)
    e2 = jnp.maximum(
        jnp.dot(e1, we2_ref[...], preferred_element_type=jnp.float32)
        + be2_ref[...], 0.0)
    e2_ref[...] = e2

    @pl.when(i == 0)
    def _():
        cs1_ref[...] = jnp.zeros_like(cs1_ref)
        cs2_ref[...] = jnp.zeros_like(cs2_ref)

    cs1_ref[...] += jnp.sum(e1, axis=0, keepdims=True)
    cs2_ref[...] += jnp.sum(e2, axis=0, keepdims=True)


def _edge_mlp_c(edges, W_e1, b_e1, W_e2, b_e2):
    full = lambda r, c: pl.BlockSpec((r, c), lambda i: (0, 0))
    return pl.pallas_call(
        _edge_mlp_c_body,
        grid=(_EGRID,),
        in_specs=[
            pl.BlockSpec((_BE, DE), lambda i: (i, 0)),
            full(DE, H1), full(1, H1), full(H1, H2), full(1, H2),
        ],
        out_specs=[
            pl.BlockSpec((_BE, H2), lambda i: (i, 0)),
            full(1, H1), full(1, H2),
        ],
        out_shape=[
            jax.ShapeDtypeStruct((E, H2), jnp.float32),
            jax.ShapeDtypeStruct((1, H1), jnp.float32),
            jax.ShapeDtypeStruct((1, H2), jnp.float32),
        ],
    )(edges, W_e1, b_e1.reshape(1, H1), W_e2, b_e2.reshape(1, H2))


_CHUNK = 80
_EPT = E // 32
_RPT = _EPT // _CHUNK
_EPC_T = E // 16
_RC = _EPC_T // _CHUNK
_NPT = N // 16
_FS = 64


_SCW = 50
_NSC = _RC // _SCW
_GB = _SCW // 2


def _sc_counts_body(rids, sids, zeros, ones, cntr, cnts,
                    tab_c, idbuf, ones_v, sem_s):
    c = lax.axis_index("c")
    s = lax.axis_index("s")

    pltpu.sync_copy(ones, ones_v)

    pltpu.sync_copy(zeros.at[:, pl.ds(0, 16)],
                    tab_c.at[pl.ds(_NPT * s, _NPT), :])
    plsc.subcore_barrier()

    def _count_pass(ids):
        def outer(m, _):
            row0 = s * _RC + m * _SCW
            pltpu.sync_copy(ids.at[pl.ds(row0, _SCW), :], idbuf)

            def inner(j, _):
                pltpu.async_copy(ones_v, tab_c.at[idbuf.at[j]], sem_s,
                                 add=True)
                return 0
            lax.fori_loop(0, _SCW, inner, 0)

            def drain(j, _):
                pltpu.make_async_copy(
                    ones_v, tab_c.at[pl.ds(0, _CHUNK), :], sem_s).wait()
                return 0
            lax.fori_loop(0, _SCW, drain, 0)
            return 0
        lax.fori_loop(0, _NSC, outer, 0)

    @pl.when(c == 0)
    def _():
        _count_pass(rids)

    @pl.when(c == 1)
    def _():
        _count_pass(sids)

    plsc.subcore_barrier()

    @pl.when(c == 0)
    def _():
        pltpu.sync_copy(tab_c.at[pl.ds(_NPT * s, _NPT), :],
                        cntr.at[pl.ds(_NPT * s, _NPT), :])

    @pl.when(c == 1)
    def _():
        pltpu.sync_copy(tab_c.at[pl.ds(_NPT * s, _NPT), :],
                        cnts.at[pl.ds(_NPT * s, _NPT), :])


def _sc_counts(rids, sids, zeros, ones):
    mesh = plsc.VectorSubcoreMesh(core_axis_name="c", subcore_axis_name="s")
    f32 = jnp.float32
    fn = pl.kernel(
        _sc_counts_body,
        out_type=[
            jax.ShapeDtypeStruct((N, 16), f32),
            jax.ShapeDtypeStruct((N, 16), f32),
        ],
        mesh=mesh,
        compiler_params=pltpu.CompilerParams(use_tc_tiling_on_sc=False),
        scratch_types=[
            pltpu.VMEM_SHARED((N, 16), f32),
            pltpu.VMEM((_SCW, _CHUNK), jnp.int32),
            pltpu.VMEM((_CHUNK, 16), f32),
            pltpu.SemaphoreType.DMA,
        ],
    )
    return fn(rids, sids, zeros, ones)


_HRC = _RC // 2


def _sc_seg_kernel(srcs, rids_r, sids_r, zeros_r):
    npass = len(srcs)

    def body(*refs):
        e_refs = refs[:npass]
        rids, sids, zeros = refs[npass:npass + 3]
        dsts = refs[npass + 3:npass + 3 + 2 * npass]
        (tab_a, tab_b, rid_v, sid_v, vc0, vc1,
         sem_v, sem_s, sem_z) = refs[npass + 3 + 2 * npass:]
        c = lax.axis_index("c")
        s = lax.axis_index("s")
        col0 = c * _FS

        def load_ids(half):
            row0 = s * _RC + half * _HRC
            pltpu.sync_copy(rids.at[pl.ds(row0, _HRC), :], rid_v)
            pltpu.sync_copy(sids.at[pl.ds(row0, _HRC), :], sid_v)

        def drain2(buf):
            pltpu.make_async_copy(buf, tab_a.at[pl.ds(0, _CHUNK), :],
                                  sem_s).wait()
            pltpu.make_async_copy(buf, tab_b.at[pl.ds(0, _CHUNK), :],
                                  sem_s).wait()

        for p in range(npass):
            src = e_refs[p]
            dst_a = dsts[2 * p]
            dst_b = dsts[2 * p + 1]

            def vload(j, buf):
                base = (s * _RC + j) * _CHUNK
                return pltpu.make_async_copy(
                    src.at[pl.ds(base, _CHUNK), pl.ds(col0, _FS)], buf,
                    sem_v)

            za = pltpu.async_copy(zeros, tab_a.at[pl.ds(_NPT * s, _NPT), :],
                                  sem_z)
            zb = pltpu.async_copy(zeros, tab_b.at[pl.ds(_NPT * s, _NPT), :],
                                  sem_z)
            load_ids(0)
            vload(0, vc0).start()
            za.wait()
            zb.wait()
            plsc.subcore_barrier()

            def outer(g, _):
                for b in (0, 1):
                    vb_, vbn = (vc0, vc1) if b == 0 else (vc1, vc0)
                    j = 2 * g + b

                    @pl.when(j > 0)
                    def _():
                        drain2(vbn)

                    if b == 1:
                        @pl.when(j == _HRC)
                        def _():
                            load_ids(1)

                    jr = lax.select(j >= _HRC, j - _HRC, j)
                    vload(j, vb_).wait()

                    @pl.when(j < _RC - 1)
                    def _():
                        vload(j + 1, vbn).start()

                    pltpu.async_copy(vb_, tab_a.at[rid_v.at[jr]], sem_s,
                                     add=True)
                    pltpu.async_copy(vb_, tab_b.at[sid_v.at[jr]], sem_s,
                                     add=True)
                return 0
            lax.fori_loop(0, _RC // 2, outer, 0)
            drain2(vc1)
            plsc.subcore_barrier()

            da = pltpu.async_copy(tab_a.at[pl.ds(_NPT * s, _NPT), :],
                                  dst_a.at[pl.ds(_NPT * s, _NPT),
                                           pl.ds(col0, _FS)], sem_z)
            db = pltpu.async_copy(tab_b.at[pl.ds(_NPT * s, _NPT), :],
                                  dst_b.at[pl.ds(_NPT * s, _NPT),
                                           pl.ds(col0, _FS)], sem_z)
            da.wait()
            db.wait()

    mesh = plsc.VectorSubcoreMesh(core_axis_name="c", subcore_axis_name="s")
    f32 = jnp.float32
    fn = pl.kernel(
        body,
        out_type=[jax.ShapeDtypeStruct((N, H2), f32)] * (2 * npass),
        mesh=mesh,
        compiler_params=pltpu.CompilerParams(use_tc_tiling_on_sc=False),
        scratch_types=[
            pltpu.VMEM_SHARED((N, _FS), f32),
            pltpu.VMEM_SHARED((N, _FS), f32),
            pltpu.VMEM((_HRC, _CHUNK), jnp.int32),
            pltpu.VMEM((_HRC, _CHUNK), jnp.int32),
            pltpu.VMEM((_CHUNK, _FS), f32),
            pltpu.VMEM((_CHUNK, _FS), f32),
            pltpu.SemaphoreType.DMA,
            pltpu.SemaphoreType.DMA,
            pltpu.SemaphoreType.DMA,
        ],
    )
    return fn(*srcs, rids_r, sids_r, zeros_r)


_BN = 1000
_NGRID = N // _BN


def _node1_body(nodes_ref, inc1a_ref, inc1b_ref, outg1a_ref, outg1b_ref,
                cntr_ref, cnts_ref,
                wn1, win1a, win1b, wout1a, wout1b, bn1,
                n1_ref, csn1_ref):
    i = pl.program_id(0)
    dot = functools.partial(jnp.dot, preferred_element_type=jnp.float32)
    cr = jnp.maximum(cntr_ref[:, 0:1], 1.0)
    cs = jnp.maximum(cnts_ref[:, 0:1], 1.0)
    n1 = jnp.maximum(
        dot(nodes_ref[...], wn1[...])
        + dot(inc1a_ref[...] / cr, win1a[...])
        + dot(inc1b_ref[...] / cr, win1b[...])
        + dot(outg1a_ref[...] / cs, wout1a[...])
        + dot(outg1b_ref[...] / cs, wout1b[...])
        + bn1[...], 0.0)
    n1_ref[...] = n1

    @pl.when(i == 0)
    def _():
        csn1_ref[...] = jnp.zeros_like(csn1_ref)

    csn1_ref[...] += jnp.sum(n1, axis=0, keepdims=True)


def _node1(nodes, inc1a, inc1b, outg1a, outg1b, cntr, cnts,
           W_n1, W_in1, W_out1, b_n1):
    full = lambda r, c: pl.BlockSpec((r, c), lambda i: (0, 0))
    blk = lambda cdim: pl.BlockSpec((_BN, cdim), lambda i: (i, 0))
    return pl.pallas_call(
        _node1_body,
        grid=(_NGRID,),
        in_specs=[
            blk(DN), blk(H2), blk(H2), blk(H2), blk(H2), blk(16), blk(16),
            full(DN, H1), full(H2, H1), full(H2, H1),
            full(H2, H1), full(H2, H1), full(1, H1),
        ],
        out_specs=[
            pl.BlockSpec((_BN, H1), lambda i: (i, 0)),
            full(1, H1),
        ],
        out_shape=[
            jax.ShapeDtypeStruct((N, H1), jnp.float32),
            jax.ShapeDtypeStruct((1, H1), jnp.float32),
        ],
    )(nodes, inc1a, inc1b, outg1a, outg1b, cntr, cnts,
      W_n1, W_in1[:H2], W_in1[H2:], W_out1[:H2], W_out1[H2:],
      b_n1.reshape(1, H1))


def _node2_body(n1_ref, inc2_ref, outg2_ref, cntr_ref, cnts_ref,
                csn1_ref, cs1_ref, cs2_ref, g_ref,
                wn2, win2, wout2, bn2, wg1, wgn1, wge1, bg1,
                wg2, wgn2, wge2, bg2,
                n2_ref, g2_ref, csn2_acc):
    i = pl.program_id(0)
    dot = functools.partial(jnp.dot, preferred_element_type=jnp.float32)
    cr = jnp.maximum(cntr_ref[:, 0:1], 1.0)
    cs = jnp.maximum(cnts_ref[:, 0:1], 1.0)
    n2 = jnp.maximum(
        dot(n1_ref[...], wn2[...]) + dot(inc2_ref[...] / cr, win2[...])
        + dot(outg2_ref[...] / cs, wout2[...]) + bn2[...], 0.0)
    n2_ref[...] = n2

    @pl.when(i == 0)
    def _():
        csn2_acc[...] = jnp.zeros_like(csn2_acc)

    csn2_acc[...] += jnp.sum(n2, axis=0, keepdims=True)

    @pl.when(i == _NGRID - 1)
    def _():
        g1 = jnp.maximum(
            dot(g_ref[...], wg1[...]) + dot(csn1_ref[...] / N, wgn1[...])
            + dot(cs1_ref[...] / E, wge1[...]) + bg1[...], 0.0)
        g2_ref[...] = jnp.maximum(
            dot(g1, wg2[...]) + dot(csn2_acc[...] / N, wgn2[...])
            + dot(cs2_ref[...] / E, wge2[...]) + bg2[...], 0.0)


def _node2(n1, inc2s, outg2s, cntr, cnts, csn1, cs1, cs2, globals_,
           W_n2, W_in2, W_out2, b_n2, W_g1, W_gn1, W_ge1, b_g1,
           W_g2, W_gn2, W_ge2, b_g2):
    full = lambda r, c: pl.BlockSpec((r, c), lambda i: (0, 0))
    blk = lambda cdim: pl.BlockSpec((_BN, cdim), lambda i: (i, 0))
    return pl.pallas_call(
        _node2_body,
        grid=(_NGRID,),
        in_specs=[
            blk(H1), blk(H2), blk(H2), blk(16), blk(16),
            full(1, H1), full(1, H1), full(1, H2), full(1, DG),
            full(H1, H2), full(H2, H2), full(H2, H2), full(1, H2),
            full(DG, H1), full(H1, H1), full(H1, H1), full(1, H1),
            full(H1, H2), full(H2, H2), full(H2, H2), full(1, H2),
        ],
        out_specs=[
            pl.BlockSpec((_BN, H2), lambda i: (i, 0)),
            full(1, H2),
        ],
        out_shape=[
            jax.ShapeDtypeStruct((N, H2), jnp.float32),
            jax.ShapeDtypeStruct((1, H2), jnp.float32),
        ],
        scratch_shapes=[
            pltpu.VMEM((1, H2), jnp.float32),
        ],
    )(n1, inc2s, outg2s, cntr, cnts, csn1, cs1, cs2,
      globals_.reshape(1, DG),
      W_n2, W_in2, W_out2, b_n2.reshape(1, H2),
      W_g1, W_gn1, W_ge1, b_g1.reshape(1, H1),
      W_g2, W_gn2, W_ge2, b_g2.reshape(1, H2))


def kernel(nodes, edges, globals_, senders, receivers,
           W_e1, b_e1, W_n1, W_in1, W_out1, b_n1, W_g1, W_gn1, W_ge1, b_g1,
           W_e2, b_e2, W_n2, W_in2, W_out2, b_n2, W_g2, W_gn2, W_ge2, b_g2):
    rids = receivers.astype(jnp.int32).reshape(E // _CHUNK, _CHUNK)
    sids = senders.astype(jnp.int32).reshape(E // _CHUNK, _CHUNK)
    zeros = jnp.zeros((_NPT, _FS), jnp.float32)
    ones = jnp.ones((_CHUNK, 16), jnp.float32)

    cntr, cnts = _sc_counts(rids, sids, zeros, ones)
    e1a = _edge_mlp_a(edges, W_e1[:, :H2], b_e1[:H2])
    inc1a, outg1a = _sc_seg_kernel([e1a], rids, sids, zeros)
    e1b = _edge_mlp_a(edges, W_e1[:, H2:], b_e1[H2:])
    e2, cs1, cs2 = _edge_mlp_c(edges, W_e1, b_e1, W_e2, b_e2)
    inc1b, outg1b, inc2s, outg2s = _sc_seg_kernel([e1b, e2], rids, sids,
                                                  zeros)
    n1, csn1 = _node1(nodes, inc1a, inc1b, outg1a, outg1b, cntr, cnts,
                      W_n1, W_in1, W_out1, b_n1)
    n2, g2 = _node2(n1, inc2s, outg2s, cntr, cnts, csn1, cs1, cs2, globals_,
                    W_n2, W_in2, W_out2, b_n2,
                    W_g1, W_gn1, W_ge1, b_g1, W_g2, W_gn2, W_ge2, b_g2)
    return (n2, e2, g2.reshape(H2))


# speedup vs baseline: 1.1101x; 1.1101x over previous
import functools

import jax
import jax.numpy as jnp
from jax import lax
from jax.experimental import pallas as pl
from jax.experimental.pallas import tpu as pltpu
from jax.experimental.pallas import tpu_sc as plsc

N = 10000
E = 320000
DN = 128
DE = 16
DG = 128
H1 = 256
H2 = 128


_BE = 2000
_EGRID = E // _BE


def _edge_mlp_a_body(edges_ref, we1a_ref, be1a_ref, e1a_ref):
    x = edges_ref[...]
    e1a_ref[...] = jnp.maximum(
        jnp.dot(x, we1a_ref[...], preferred_element_type=jnp.float32)
        + be1a_ref[...], 0.0)


def _edge_mlp_a(edges, W_e1a, b_e1a):
    full = lambda r, c: pl.BlockSpec((r, c), lambda i: (0, 0))
    return pl.pallas_call(
        _edge_mlp_a_body,
        grid=(_EGRID,),
        in_specs=[
            pl.BlockSpec((_BE, DE), lambda i: (i, 0)),
            full(DE, H2), full(1, H2),
        ],
        out_specs=[pl.BlockSpec((_BE, H2), lambda i: (i, 0))],
        out_shape=[jax.ShapeDtypeStruct((E, H2), jnp.float32)],
    )(edges, W_e1a, b_e1a.reshape(1, H2))[0]


def _edge_mlp_b_body(edges_ref, we1_ref, be1_ref, we2_ref, be2_ref,
                     e1b_ref, e2_ref, cs1_ref, cs2_ref):
    i = pl.program_id(0)
    x = edges_ref[...]
    e1 = jnp.maximum(
        jnp.dot(x, we1_ref[...], preferred_element_type=jnp.float32)
        + be1_ref[...], 0.0)
    e2 = jnp.maximum(
        jnp.dot(e1, we2_ref[...], preferred_element_type=jnp.float32)
        + be2_ref[...], 0.0)
    e1b_ref[...] = e1[:, H2:]
    e2_ref[...] = e2

    @pl.when(i == 0)
    def _():
        cs1_ref[...] = jnp.zeros_like(cs1_ref)
        cs2_ref[...] = jnp.zeros_like(cs2_ref)

    cs1_ref[...] += jnp.sum(e1, axis=0, keepdims=True)
    cs2_ref[...] += jnp.sum(e2, axis=0, keepdims=True)


def _edge_mlp_b(edges, W_e1, b_e1, W_e2, b_e2):
    full = lambda r, c: pl.BlockSpec((r, c), lambda i: (0, 0))
    return pl.pallas_call(
        _edge_mlp_b_body,
        grid=(_EGRID,),
        in_specs=[
            pl.BlockSpec((_BE, DE), lambda i: (i, 0)),
            full(DE, H1), full(1, H1), full(H1, H2), full(1, H2),
        ],
        out_specs=[
            pl.BlockSpec((_BE, H2), lambda i: (i, 0)),
            pl.BlockSpec((_BE, H2), lambda i: (i, 0)),
            full(1, H1), full(1, H2),
        ],
        out_shape=[
            jax.ShapeDtypeStruct((E, H2), jnp.float32),
            jax.ShapeDtypeStruct((E, H2), jnp.float32),
            jax.ShapeDtypeStruct((1, H1), jnp.float32),
            jax.ShapeDtypeStruct((1, H2), jnp.float32),
        ],
    )(edges, W_e1, b_e1.reshape(1, H1), W_e2, b_e2.reshape(1, H2))


_CHUNK = 80
_EPT = E // 32
_RPT = _EPT // _CHUNK
_EPC_T = E // 16
_RC = _EPC_T // _CHUNK
_NPT = N // 16
_FS = 64


_SCW = 50
_NSC = _RC // _SCW
_GB = _SCW // 2


def _sc_counts_body(rids, sids, zeros, ones, cntr, cnts,
                    tab_c, idbuf, ones_v, sem_s):
    c = lax.axis_index("c")
    s = lax.axis_index("s")

    pltpu.sync_copy(ones, ones_v)

    pltpu.sync_copy(zeros.at[:, pl.ds(0, 16)],
                    tab_c.at[pl.ds(_NPT * s, _NPT), :])
    plsc.subcore_barrier()

    def _count_pass(ids):
        def outer(m, _):
            row0 = s * _RC + m * _SCW
            pltpu.sync_copy(ids.at[pl.ds(row0, _SCW), :], idbuf)

            def inner(j, _):
                pltpu.async_copy(ones_v, tab_c.at[idbuf.at[j]], sem_s,
                                 add=True)
                return 0
            lax.fori_loop(0, _SCW, inner, 0)

            def drain(j, _):
                pltpu.make_async_copy(
                    ones_v, tab_c.at[pl.ds(0, _CHUNK), :], sem_s).wait()
                return 0
            lax.fori_loop(0, _SCW, drain, 0)
            return 0
        lax.fori_loop(0, _NSC, outer, 0)

    @pl.when(c == 0)
    def _():
        _count_pass(rids)

    @pl.when(c == 1)
    def _():
        _count_pass(sids)

    plsc.subcore_barrier()

    @pl.when(c == 0)
    def _():
        pltpu.sync_copy(tab_c.at[pl.ds(_NPT * s, _NPT), :],
                        cntr.at[pl.ds(_NPT * s, _NPT), :])

    @pl.when(c == 1)
    def _():
        pltpu.sync_copy(tab_c.at[pl.ds(_NPT * s, _NPT), :],
                        cnts.at[pl.ds(_NPT * s, _NPT), :])


def _sc_counts(rids, sids, zeros, ones):
    mesh = plsc.VectorSubcoreMesh(core_axis_name="c", subcore_axis_name="s")
    f32 = jnp.float32
    fn = pl.kernel(
        _sc_counts_body,
        out_type=[
            jax.ShapeDtypeStruct((N, 16), f32),
            jax.ShapeDtypeStruct((N, 16), f32),
        ],
        mesh=mesh,
        compiler_params=pltpu.CompilerParams(use_tc_tiling_on_sc=False),
        scratch_types=[
            pltpu.VMEM_SHARED((N, 16), f32),
            pltpu.VMEM((_SCW, _CHUNK), jnp.int32),
            pltpu.VMEM((_CHUNK, 16), f32),
            pltpu.SemaphoreType.DMA,
        ],
    )
    return fn(rids, sids, zeros, ones)


_HRC = _RC // 2


def _sc_seg_kernel(srcs, rids_r, sids_r, zeros_r):
    npass = len(srcs)

    def body(*refs):
        e_refs = refs[:npass]
        rids, sids, zeros = refs[npass:npass + 3]
        dsts = refs[npass + 3:npass + 3 + 2 * npass]
        (tab_a, tab_b, rid_v, sid_v, vc0, vc1,
         sem_v, sem_s, sem_z) = refs[npass + 3 + 2 * npass:]
        c = lax.axis_index("c")
        s = lax.axis_index("s")
        col0 = c * _FS

        def load_ids(half):
            row0 = s * _RC + half * _HRC
            pltpu.sync_copy(rids.at[pl.ds(row0, _HRC), :], rid_v)
            pltpu.sync_copy(sids.at[pl.ds(row0, _HRC), :], sid_v)

        def drain2(buf):
            pltpu.make_async_copy(buf, tab_a.at[pl.ds(0, _CHUNK), :],
                                  sem_s).wait()
            pltpu.make_async_copy(buf, tab_b.at[pl.ds(0, _CHUNK), :],
                                  sem_s).wait()

        for p in range(npass):
            src = e_refs[p]
            dst_a = dsts[2 * p]
            dst_b = dsts[2 * p + 1]

            def vload(j, buf):
                base = (s * _RC + j) * _CHUNK
                return pltpu.make_async_copy(
                    src.at[pl.ds(base, _CHUNK), pl.ds(col0, _FS)], buf,
                    sem_v)

            za = pltpu.async_copy(zeros, tab_a.at[pl.ds(_NPT * s, _NPT), :],
                                  sem_z)
            zb = pltpu.async_copy(zeros, tab_b.at[pl.ds(_NPT * s, _NPT), :],
                                  sem_z)
            load_ids(0)
            vload(0, vc0).start()
            za.wait()
            zb.wait()
            plsc.subcore_barrier()

            def outer(g, _):
                for b in (0, 1):
                    vb_, vbn = (vc0, vc1) if b == 0 else (vc1, vc0)
                    j = 2 * g + b

                    @pl.when(j > 0)
                    def _():
                        drain2(vbn)

                    if b == 1:
                        @pl.when(j == _HRC)
                        def _():
                            load_ids(1)

                    jr = lax.select(j >= _HRC, j - _HRC, j)
                    vload(j, vb_).wait()

                    @pl.when(j < _RC - 1)
                    def _():
                        vload(j + 1, vbn).start()

                    pltpu.async_copy(vb_, tab_a.at[rid_v.at[jr]], sem_s,
                                     add=True)
                    pltpu.async_copy(vb_, tab_b.at[sid_v.at[jr]], sem_s,
                                     add=True)
                return 0
            lax.fori_loop(0, _RC // 2, outer, 0)
            drain2(vc1)
            plsc.subcore_barrier()

            da = pltpu.async_copy(tab_a.at[pl.ds(_NPT * s, _NPT), :],
                                  dst_a.at[pl.ds(_NPT * s, _NPT),
                                           pl.ds(col0, _FS)], sem_z)
            db = pltpu.async_copy(tab_b.at[pl.ds(_NPT * s, _NPT), :],
                                  dst_b.at[pl.ds(_NPT * s, _NPT),
                                           pl.ds(col0, _FS)], sem_z)
            da.wait()
            db.wait()

    mesh = plsc.VectorSubcoreMesh(core_axis_name="c", subcore_axis_name="s")
    f32 = jnp.float32
    fn = pl.kernel(
        body,
        out_type=[jax.ShapeDtypeStruct((N, H2), f32)] * (2 * npass),
        mesh=mesh,
        compiler_params=pltpu.CompilerParams(use_tc_tiling_on_sc=False),
        scratch_types=[
            pltpu.VMEM_SHARED((N, _FS), f32),
            pltpu.VMEM_SHARED((N, _FS), f32),
            pltpu.VMEM((_HRC, _CHUNK), jnp.int32),
            pltpu.VMEM((_HRC, _CHUNK), jnp.int32),
            pltpu.VMEM((_CHUNK, _FS), f32),
            pltpu.VMEM((_CHUNK, _FS), f32),
            pltpu.SemaphoreType.DMA,
            pltpu.SemaphoreType.DMA,
            pltpu.SemaphoreType.DMA,
        ],
    )
    return fn(*srcs, rids_r, sids_r, zeros_r)


_BN = 1000
_NGRID = N // _BN


def _node_body(nodes_ref, inc1a_ref, inc1b_ref, outg1a_ref, outg1b_ref,
               inc2_ref, outg2_ref,
               cntr_ref, cnts_ref, cs1_ref, cs2_ref, g_ref,
               wn1, win1a, win1b, wout1a, wout1b, bn1,
               wg1, wgn1, wge1, bg1,
               wn2, win2, wout2, bn2, wg2, wgn2, wge2, bg2,
               n2_ref, g2_ref, csn1_acc, csn2_acc):
    i = pl.program_id(0)
    dot = functools.partial(jnp.dot, preferred_element_type=jnp.float32)
    cr = jnp.maximum(cntr_ref[:, 0:1], 1.0)
    cs = jnp.maximum(cnts_ref[:, 0:1], 1.0)
    n1 = jnp.maximum(
        dot(nodes_ref[...], wn1[...])
        + dot(inc1a_ref[...] / cr, win1a[...])
        + dot(inc1b_ref[...] / cr, win1b[...])
        + dot(outg1a_ref[...] / cs, wout1a[...])
        + dot(outg1b_ref[...] / cs, wout1b[...])
        + bn1[...], 0.0)
    n2 = jnp.maximum(
        dot(n1, wn2[...]) + dot(inc2_ref[...] / cr, win2[...])
        + dot(outg2_ref[...] / cs, wout2[...]) + bn2[...], 0.0)
    n2_ref[...] = n2

    @pl.when(i == 0)
    def _():
        csn1_acc[...] = jnp.zeros_like(csn1_acc)
        csn2_acc[...] = jnp.zeros_like(csn2_acc)

    csn1_acc[...] += jnp.sum(n1, axis=0, keepdims=True)
    csn2_acc[...] += jnp.sum(n2, axis=0, keepdims=True)

    @pl.when(i == _NGRID - 1)
    def _():
        g1 = jnp.maximum(
            dot(g_ref[...], wg1[...]) + dot(csn1_acc[...] / N, wgn1[...])
            + dot(cs1_ref[...] / E, wge1[...]) + bg1[...], 0.0)
        g2_ref[...] = jnp.maximum(
            dot(g1, wg2[...]) + dot(csn2_acc[...] / N, wgn2[...])
            + dot(cs2_ref[...] / E, wge2[...]) + bg2[...], 0.0)


def _node_global(nodes, inc1a, inc1b, outg1a, outg1b, inc2s, outg2s,
                 cntr, cnts, cs1, cs2, globals_,
                 W_n1, W_in1, W_out1, b_n1, W_g1, W_gn1, W_ge1, b_g1,
                 W_n2, W_in2, W_out2, b_n2, W_g2, W_gn2, W_ge2, b_g2):
    full = lambda r, c: pl.BlockSpec((r, c), lambda i: (0, 0))
    blk = lambda cdim: pl.BlockSpec((_BN, cdim), lambda i: (i, 0))
    return pl.pallas_call(
        _node_body,
        grid=(_NGRID,),
        in_specs=[
            blk(DN), blk(H2), blk(H2), blk(H2), blk(H2), blk(H2), blk(H2),
            blk(16), blk(16),
            full(1, H1), full(1, H2), full(1, DG),
            full(DN, H1), full(H2, H1), full(H2, H1),
            full(H2, H1), full(H2, H1), full(1, H1),
            full(DG, H1), full(H1, H1), full(H1, H1), full(1, H1),
            full(H1, H2), full(H2, H2), full(H2, H2), full(1, H2),
            full(H1, H2), full(H2, H2), full(H2, H2), full(1, H2),
        ],
        out_specs=[
            pl.BlockSpec((_BN, H2), lambda i: (i, 0)),
            full(1, H2),
        ],
        out_shape=[
            jax.ShapeDtypeStruct((N, H2), jnp.float32),
            jax.ShapeDtypeStruct((1, H2), jnp.float32),
        ],
        scratch_shapes=[
            pltpu.VMEM((1, H1), jnp.float32),
            pltpu.VMEM((1, H2), jnp.float32),
        ],
    )(nodes, inc1a, inc1b, outg1a, outg1b, inc2s, outg2s, cntr, cnts,
      cs1, cs2, globals_.reshape(1, DG),
      W_n1, W_in1[:H2], W_in1[H2:], W_out1[:H2], W_out1[H2:],
      b_n1.reshape(1, H1),
      W_g1, W_gn1, W_ge1, b_g1.reshape(1, H1),
      W_n2, W_in2, W_out2, b_n2.reshape(1, H2),
      W_g2, W_gn2, W_ge2, b_g2.reshape(1, H2))


def kernel(nodes, edges, globals_, senders, receivers,
           W_e1, b_e1, W_n1, W_in1, W_out1, b_n1, W_g1, W_gn1, W_ge1, b_g1,
           W_e2, b_e2, W_n2, W_in2, W_out2, b_n2, W_g2, W_gn2, W_ge2, b_g2):
    rids = receivers.astype(jnp.int32).reshape(E // _CHUNK, _CHUNK)
    sids = senders.astype(jnp.int32).reshape(E // _CHUNK, _CHUNK)
    zeros = jnp.zeros((_NPT, _FS), jnp.float32)
    ones = jnp.ones((_CHUNK, 16), jnp.float32)

    cntr, cnts = _sc_counts(rids, sids, zeros, ones)
    e1a = _edge_mlp_a(edges, W_e1[:, :H2], b_e1[:H2])
    inc1a, outg1a = _sc_seg_kernel([e1a], rids, sids, zeros)
    e1b, e2, cs1, cs2 = _edge_mlp_b(edges, W_e1, b_e1, W_e2, b_e2)
    inc1b, outg1b, inc2s, outg2s = _sc_seg_kernel([e1b, e2], rids, sids,
                                                  zeros)
    n2, g2 = _node_global(
        nodes, inc1a, inc1b, outg1a, outg1b, inc2s, outg2s, cntr, cnts,
        cs1, cs2, globals_,
        W_n1, W_in1, W_out1, b_n1, W_g1, W_gn1, W_ge1, b_g1,
        W_n2, W_in2, W_out2, b_n2, W_g2, W_gn2, W_ge2, b_g2)
    return (n2, e2, g2.reshape(H2))
